# baseline (speedup 1.0000x reference)
import jax
import jax.numpy as jnp
from jax.experimental import pallas as pl
from jax.experimental.pallas import tpu as pltpu

_TILE = 512


def _mlp_kernel(x_ref, w1_ref, b1_ref, w2_ref, b2_ref, w3_ref, b3_ref, o_ref):
    x = x_ref[...].astype(jnp.bfloat16)
    h1 = jnp.dot(x, w1_ref[...], preferred_element_type=jnp.float32)
    h1 = jnp.maximum(h1 + b1_ref[...], 0.0).astype(jnp.bfloat16)
    h2 = jnp.dot(h1, w2_ref[...], preferred_element_type=jnp.float32)
    h2 = jnp.maximum(h2 + b2_ref[...], 0.0).astype(jnp.bfloat16)
    out = jnp.dot(h2, w3_ref[...], preferred_element_type=jnp.float32)
    o_ref[...] = out + b3_ref[...]


def kernel(field, W1, b1, W2, b2, W3, b3):
    B = field.shape[0]
    in_dim = field.shape[1] * field.shape[2]
    hid1 = W1.shape[0]
    hid2 = W2.shape[0]

    x = field.reshape(B, in_dim)
    w1t = W1.T.astype(jnp.bfloat16)
    w2t = W2.T.astype(jnp.bfloat16)
    w3t = W3.T.astype(jnp.bfloat16)
    b1r = b1.reshape(1, hid1)
    b2r = b2.reshape(1, hid2)
    b3r = b3.reshape(1, 1)

    grid = (B // _TILE,)
    out = pl.pallas_call(
        _mlp_kernel,
        grid=grid,
        in_specs=[
            pl.BlockSpec((_TILE, in_dim), lambda i: (i, 0)),
            pl.BlockSpec((in_dim, hid1), lambda i: (0, 0)),
            pl.BlockSpec((1, hid1), lambda i: (0, 0)),
            pl.BlockSpec((hid1, hid2), lambda i: (0, 0)),
            pl.BlockSpec((1, hid2), lambda i: (0, 0)),
            pl.BlockSpec((hid2, 1), lambda i: (0, 0)),
            pl.BlockSpec((1, 1), lambda i: (0, 0)),
        ],
        out_specs=pl.BlockSpec((_TILE, 1), lambda i: (i, 0)),
        out_shape=jax.ShapeDtypeStruct((B, 1), jnp.float32),
    )(x, w1t, b1r, w2t, b2r, w3t, b3r)
    return out


# speedup vs baseline: 1.0148x; 1.0148x over previous
import jax
import jax.numpy as jnp
from jax.experimental import pallas as pl
from jax.experimental.pallas import tpu as pltpu

_TILE = 512

_DN_T = (((1,), (1,)), ((), ()))


def _mlp_kernel(x_ref, w1_ref, b1_ref, w2_ref, b2_ref, w3_ref, b3_ref,
                o_ref, w1bf_ref):
    @pl.when(pl.program_id(0) == 0)
    def _():
        w1bf_ref[...] = w1_ref[...].astype(jnp.bfloat16)

    x = x_ref[...].astype(jnp.bfloat16)
    h1 = jax.lax.dot_general(x, w1bf_ref[...], _DN_T,
                             preferred_element_type=jnp.float32)
    h1 = jnp.maximum(h1 + b1_ref[...], 0.0).astype(jnp.bfloat16)
    h2 = jnp.dot(h1, w2_ref[...].astype(jnp.bfloat16),
                 preferred_element_type=jnp.float32)
    h2 = jnp.maximum(h2 + b2_ref[...], 0.0).astype(jnp.bfloat16)
    out = jnp.dot(h2, w3_ref[...].astype(jnp.bfloat16),
                  preferred_element_type=jnp.float32)
    o_ref[...] = out + b3_ref[...]


def kernel(field, W1, b1, W2, b2, W3, b3):
    B = field.shape[0]
    in_dim = field.shape[1] * field.shape[2]
    hid1 = W1.shape[0]
    hid2 = W2.shape[0]

    x = field.reshape(B, in_dim)
    w2t = W2.T
    w3t = W3.T
    b1r = b1.reshape(1, hid1)
    b2r = b2.reshape(1, hid2)
    b3r = b3.reshape(1, 1)

    grid = (B // _TILE,)
    out = pl.pallas_call(
        _mlp_kernel,
        grid=grid,
        in_specs=[
            pl.BlockSpec((_TILE, in_dim), lambda i: (i, 0)),
            pl.BlockSpec((hid1, in_dim), lambda i: (0, 0)),
            pl.BlockSpec((1, hid1), lambda i: (0, 0)),
            pl.BlockSpec((hid1, hid2), lambda i: (0, 0)),
            pl.BlockSpec((1, hid2), lambda i: (0, 0)),
            pl.BlockSpec((hid2, 1), lambda i: (0, 0)),
            pl.BlockSpec((1, 1), lambda i: (0, 0)),
        ],
        out_specs=pl.BlockSpec((_TILE, 1), lambda i: (i, 0)),
        out_shape=jax.ShapeDtypeStruct((B, 1), jnp.float32),
        scratch_shapes=[pltpu.VMEM((hid1, in_dim), jnp.bfloat16)],
    )(x, W1, b1r, w2t, b2r, w3t, b3r)
    return out


# speedup vs baseline: 1.3717x; 1.3517x over previous
import jax
import jax.numpy as jnp
from jax.experimental import pallas as pl
from jax.experimental.pallas import tpu as pltpu

_TILE = 512

_DN_T = (((1,), (1,)), ((), ()))


def _mlp_kernel(x_ref, w1_ref, b1_ref, w2_ref, b2_ref, w3_ref, b3_ref,
                o_ref, w1bf_ref):
    @pl.when(pl.program_id(0) == 0)
    def _():
        w1bf_ref[...] = w1_ref[...].astype(jnp.bfloat16)

    xt = x_ref[...]
    x = xt.astype(jnp.bfloat16).reshape(xt.shape[0], -1)
    h1 = jax.lax.dot_general(x, w1bf_ref[...], _DN_T,
                             preferred_element_type=jnp.float32)
    h1 = jnp.maximum(h1 + b1_ref[...], 0.0).astype(jnp.bfloat16)
    h2 = jnp.dot(h1, w2_ref[...].astype(jnp.bfloat16),
                 preferred_element_type=jnp.float32)
    h2 = jnp.maximum(h2 + b2_ref[...], 0.0).astype(jnp.bfloat16)
    out = jnp.dot(h2, w3_ref[...].astype(jnp.bfloat16),
                  preferred_element_type=jnp.float32)
    o_ref[...] = out + b3_ref[...]


def kernel(field, W1, b1, W2, b2, W3, b3):
    B = field.shape[0]
    in_dim = field.shape[1] * field.shape[2]
    hid1 = W1.shape[0]
    hid2 = W2.shape[0]

    nf, nl = field.shape[1], field.shape[2]
    w2t = W2.T
    w3t = W3.T
    b1r = b1.reshape(1, hid1)
    b2r = b2.reshape(1, hid2)
    b3r = b3.reshape(1, 1)

    grid = (B // _TILE,)
    out = pl.pallas_call(
        _mlp_kernel,
        grid=grid,
        in_specs=[
            pl.BlockSpec((_TILE, nf, nl), lambda i: (i, 0, 0)),
            pl.BlockSpec((hid1, in_dim), lambda i: (0, 0)),
            pl.BlockSpec((1, hid1), lambda i: (0, 0)),
            pl.BlockSpec((hid1, hid2), lambda i: (0, 0)),
            pl.BlockSpec((1, hid2), lambda i: (0, 0)),
            pl.BlockSpec((hid2, 1), lambda i: (0, 0)),
            pl.BlockSpec((1, 1), lambda i: (0, 0)),
        ],
        out_specs=pl.BlockSpec((_TILE, 1), lambda i: (i, 0)),
        out_shape=jax.ShapeDtypeStruct((B, 1), jnp.float32),
        scratch_shapes=[pltpu.VMEM((hid1, in_dim), jnp.bfloat16)],
    )(field, W1, b1r, w2t, b2r, w3t, b3r)
    return out
